# Initial kernel scaffold; baseline (speedup 1.0000x reference)
import jax
import jax.numpy as jnp
from jax.experimental import pallas as pl


def kernel(x, edge_index, edge_attr, W1, b1, W2, b2, Wskip, gamma, beta):
    raise NotImplementedError("write your pallas kernel here")


# speedup vs baseline: 2.1470x; 2.1470x over previous
import functools

import jax
import jax.numpy as jnp
from jax import lax
from jax.experimental import pallas as pl
from jax.experimental.pallas import tpu as pltpu
from jax.experimental.pallas import tpu_sc as plsc

_N = 10000
_E = 320000
_D = 128
_DE = 16
_H = 128

_NC = 2
_NS = 16
_NW = _NC * _NS
_L = 16

_CH = 64
_CPT = 158
_EP = _NW * _CPT * _CH
_NP = 10240
_RPS = _NP // _NS

_NBLK = 1000
_EBLK = 2048


def _node_pre_body(x_ref, w_ref, o_ref):
    o_ref[...] = jnp.dot(x_ref[...], w_ref[...],
                         preferred_element_type=jnp.float32)


def _edge_pre_body(ea_ref, w_ref, b_ref, o_ref):
    o_ref[...] = jnp.dot(ea_ref[...], w_ref[...],
                         preferred_element_type=jnp.float32) + b_ref[...]


def _post_body(s_ref, cnt_ref, skip_ref, w2_ref, b2_ref, g_ref, be_ref,
               o_ref):
    s = s_ref[0] + s_ref[1]
    cnt = cnt_ref[:, 0:1] + cnt_ref[:, 1:2]
    agg = jnp.dot(s, w2_ref[...], preferred_element_type=jnp.float32)
    agg = (agg + cnt * b2_ref[...]) / jnp.maximum(cnt, 1.0)
    out = agg + skip_ref[...]
    mu = jnp.mean(out, axis=-1, keepdims=True)
    var = jnp.mean((out - mu) ** 2, axis=-1, keepdims=True)
    o_ref[...] = (out - mu) * lax.rsqrt(var + 1e-5) * g_ref[...] + be_ref[...]


def _sc_edge_body(dst_hbm, src_hbm, c_hbm, a_hbm, b_hbm, z2_hbm, z1_hbm,
                  out_s_hbm, out_c_hbm,
                  dstv, srcv, av, bv, cv, onesv, s_sh, cnt_sh, sema, semb):
    cid = lax.axis_index("c")
    sid = lax.axis_index("s")
    wid = sid * _NC + cid

    my_rows = pl.ds(sid * _RPS, _RPS)
    pltpu.sync_copy(z2_hbm, s_sh.at[my_rows])
    pltpu.sync_copy(z1_hbm, cnt_sh.at[my_rows])

    for i in range(_CH // _L):
        onesv[pl.ds(i * _L, _L)] = jnp.full((_L,), 1.0, jnp.float32)

    plsc.subcore_barrier()

    def chunk_body(g, carry):
        base = (wid * _CPT + g) * _CH
        pltpu.sync_copy(dst_hbm.at[pl.ds(base, _CH)], dstv)
        pltpu.sync_copy(src_hbm.at[pl.ds(base, _CH)], srcv)
        ca = pltpu.async_copy(a_hbm.at[dstv], av, sema)
        cb = pltpu.async_copy(b_hbm.at[srcv], bv, semb)
        pltpu.sync_copy(c_hbm.at[pl.ds(base, _CH)], cv)
        ca.wait()
        cb.wait()

        def row_body(r, rc):
            for j in range(_D // _L):
                sl = pl.ds(j * _L, _L)
                cv[r, sl] = jnp.maximum(av[r, sl] + bv[r, sl] + cv[r, sl],
                                        0.0)
            return rc

        lax.fori_loop(0, _CH, row_body, 0, unroll=2)
        pltpu.sync_copy(cv, s_sh.at[dstv], add=True)
        pltpu.sync_copy(onesv, cnt_sh.at[dstv], add=True)
        return carry

    lax.fori_loop(0, _CPT, chunk_body, 0)

    plsc.subcore_barrier()
    pltpu.sync_copy(s_sh.at[my_rows], out_s_hbm.at[cid, my_rows])
    pltpu.sync_copy(cnt_sh.at[my_rows], out_c_hbm.at[cid, 0, my_rows])


def _build(interpret=False):
    node_pre = pl.pallas_call(
        _node_pre_body,
        grid=(_N // _NBLK,),
        in_specs=[
            pl.BlockSpec((_NBLK, _D), lambda i: (i, 0)),
            pl.BlockSpec((_D, 3 * _H), lambda i: (0, 0)),
        ],
        out_specs=pl.BlockSpec((_NBLK, 3 * _H), lambda i: (i, 0)),
        out_shape=jax.ShapeDtypeStruct((_N, 3 * _H), jnp.float32),
        interpret=interpret,
    )

    edge_pre = pl.pallas_call(
        _edge_pre_body,
        grid=(_EP // _EBLK,),
        in_specs=[
            pl.BlockSpec((_EBLK, _DE), lambda i: (i, 0)),
            pl.BlockSpec((_DE, _H), lambda i: (0, 0)),
            pl.BlockSpec((1, _H), lambda i: (0, 0)),
        ],
        out_specs=pl.BlockSpec((_EBLK, _H), lambda i: (i, 0)),
        out_shape=jax.ShapeDtypeStruct((_EP, _H), jnp.float32),
        interpret=interpret,
    )

    mesh = plsc.VectorSubcoreMesh(core_axis_name="c", subcore_axis_name="s")
    sc_edge = pl.kernel(
        _sc_edge_body,
        out_type=(
            jax.ShapeDtypeStruct((_NC, _NP, _D), jnp.float32),
            jax.ShapeDtypeStruct((_NC, 1, _NP), jnp.float32),
        ),
        mesh=mesh,
        scratch_types=[
            pltpu.VMEM((_CH,), jnp.int32),
            pltpu.VMEM((_CH,), jnp.int32),
            pltpu.VMEM((_CH, _D), jnp.float32),
            pltpu.VMEM((_CH, _D), jnp.float32),
            pltpu.VMEM((_CH, _D), jnp.float32),
            pltpu.VMEM((_CH,), jnp.float32),
            pltpu.VMEM_SHARED((_NP, _D), jnp.float32),
            pltpu.VMEM_SHARED((_NP,), jnp.float32),
            pltpu.SemaphoreType.DMA,
            pltpu.SemaphoreType.DMA,
        ],
        interpret=interpret,
    )

    post = pl.pallas_call(
        _post_body,
        grid=(_N // _NBLK,),
        in_specs=[
            pl.BlockSpec((_NC, _NBLK, _D), lambda i: (0, i, 0)),
            pl.BlockSpec((_NBLK, _NC), lambda i: (i, 0)),
            pl.BlockSpec((_NBLK, _D), lambda i: (i, 0)),
            pl.BlockSpec((_D, _D), lambda i: (0, 0)),
            pl.BlockSpec((1, _D), lambda i: (0, 0)),
            pl.BlockSpec((1, _D), lambda i: (0, 0)),
            pl.BlockSpec((1, _D), lambda i: (0, 0)),
        ],
        out_specs=pl.BlockSpec((_NBLK, _D), lambda i: (i, 0)),
        out_shape=jax.ShapeDtypeStruct((_N, _D), jnp.float32),
        interpret=interpret,
    )

    @jax.jit
    def run(x, edge_index, edge_attr, W1, b1, W2, b2, Wskip, gamma, beta):
        dst = edge_index[0]
        src = edge_index[1]

        wn = jnp.concatenate([W1[:_D], W1[_D:2 * _D], Wskip], axis=1)
        nodes = node_pre(x, wn)
        a_tab = nodes[:, :_H]
        b_tab = nodes[:, _H:2 * _H]
        skip = nodes[:, 2 * _H:]
        pad_n = ((0, _NP - _N), (0, 0))
        a_tab = jnp.pad(a_tab, pad_n)
        b_tab = jnp.pad(b_tab, pad_n)

        dst_p = jnp.concatenate(
            [dst, jnp.full((_EP - _E,), _N, jnp.int32)])
        src_p = jnp.concatenate(
            [src, jnp.zeros((_EP - _E,), jnp.int32)])
        ea_p = jnp.pad(edge_attr, ((0, _EP - _E), (0, 0)))
        c_tab = edge_pre(ea_p, W1[2 * _D:], b1.reshape(1, _H))

        z2 = jnp.zeros((_RPS, _D), jnp.float32)
        z1 = jnp.zeros((_RPS,), jnp.float32)
        s_out, cnt_out = sc_edge(dst_p, src_p, c_tab, a_tab, b_tab, z2, z1)

        cnt_t = jnp.swapaxes(cnt_out.reshape(_NC, _NP), 0, 1)
        out = post(s_out, cnt_t, skip, W2, b2.reshape(1, _D),
                   gamma.reshape(1, _D), beta.reshape(1, _D))
        return out

    return run


_impl = _build()


def kernel(x, edge_index, edge_attr, W1, b1, W2, b2, Wskip, gamma, beta):
    return _impl(x, edge_index, edge_attr, W1, b1, W2, b2, Wskip, gamma,
                 beta)


# speedup vs baseline: 3.9151x; 1.8235x over previous
import functools

import jax
import jax.numpy as jnp
from jax import lax
from jax.experimental import pallas as pl
from jax.experimental.pallas import tpu as pltpu
from jax.experimental.pallas import tpu_sc as plsc

_N = 10000
_E = 320000
_D = 128
_DE = 16
_H = 128

_NC = 2
_NS = 16
_NW = _NC * _NS
_L = 16

_CH = 64
_CPT = 158
_EP = _NW * _CPT * _CH
_NP = 10112
_RPS = _NP // _NS
_NPC = 10240
_CPS = _NPC // _NS

_NBLK = 1000
_EBLK = 2048


def _node_pre_body(x_ref, w_ref, o_ref):
    o_ref[...] = jnp.dot(x_ref[...], w_ref[...],
                         preferred_element_type=jnp.float32)


def _edge_pre_body(ea_ref, w_ref, b_ref, o_ref):
    o_ref[...] = jnp.dot(ea_ref[...], w_ref[...],
                         preferred_element_type=jnp.float32) + b_ref[...]


def _post_body(s_ref, cnt_ref, skip_ref, w2_ref, b2_ref, g_ref, be_ref,
               o_ref):
    s = s_ref[0] + s_ref[1]
    cnt = cnt_ref[:, 0:1] + cnt_ref[:, 1:2]
    agg = jnp.dot(s, w2_ref[...], preferred_element_type=jnp.float32)
    agg = (agg + cnt * b2_ref[...]) / jnp.maximum(cnt, 1.0)
    out = agg + skip_ref[...]
    mu = jnp.mean(out, axis=-1, keepdims=True)
    var = jnp.mean((out - mu) ** 2, axis=-1, keepdims=True)
    o_ref[...] = (out - mu) * lax.rsqrt(var + 1e-5) * g_ref[...] + be_ref[...]


def _sc_edge_body(dst_hbm, src_hbm, c_hbm, a_hbm, b_hbm, z2_hbm, z1_hbm,
                  out_s_hbm, out_c_hbm,
                  dstv0, srcv0, dstv1, srcv1, av0, bv0, av1, bv1, cv0,
                  onesv, s_sh, cnt_sh,
                  sema0, semb0, sema1, semb1, semc0):
    cid = lax.axis_index("c")
    sid = lax.axis_index("s")
    tbase = (sid * _NC + cid) * _CPT

    my_rows = pl.ds(sid * _RPS, _RPS)
    pltpu.sync_copy(z2_hbm, s_sh.at[my_rows])
    pltpu.sync_copy(z1_hbm, cnt_sh.at[pl.ds(sid * _CPS, _CPS)])

    for i in range(_CH // _L):
        onesv[pl.ds(i * _L, _L)] = jnp.full((_L,), 1.0, jnp.float32)

    plsc.subcore_barrier()

    sets = ((dstv0, srcv0, av0, bv0, sema0, semb0),
            (dstv1, srcv1, av1, bv1, sema1, semb1))

    def cbase(g):
        return (tbase + g) * _CH

    def fetch(g, s):
        dstv, srcv, av, bv, sa, sb = s
        base = cbase(g)
        pltpu.sync_copy(dst_hbm.at[pl.ds(base, _CH)], dstv)
        pltpu.sync_copy(src_hbm.at[pl.ds(base, _CH)], srcv)
        pltpu.async_copy(a_hbm.at[dstv], av, sa)
        pltpu.async_copy(b_hbm.at[srcv], bv, sb)

    def process(g, p):
        dstv, srcv, av, bv, sa, sb = sets[p]
        fetch(jnp.minimum(g + 1, _CPT - 1), sets[1 - p])
        pltpu.make_async_copy(a_hbm.at[dstv], av, sa).wait()
        pltpu.make_async_copy(b_hbm.at[srcv], bv, sb).wait()
        pltpu.make_async_copy(c_hbm.at[pl.ds(0, _CH)], cv0, semc0).wait()

        @plsc.parallel_loop(0, _CH, unroll=4)
        def _(r):
            for j in range(_D // _L):
                sl = pl.ds(j * _L, _L)
                cv0[r, sl] = jnp.maximum(av[r, sl] + bv[r, sl] + cv0[r, sl],
                                         0.0)

        pltpu.sync_copy(cv0, s_sh.at[dstv], add=True)
        pltpu.sync_copy(onesv, cnt_sh.at[dstv], add=True)
        pltpu.async_copy(c_hbm.at[pl.ds(cbase(jnp.minimum(g + 1, _CPT - 1)),
                                        _CH)], cv0, semc0)

    fetch(0, sets[0])
    pltpu.async_copy(c_hbm.at[pl.ds(cbase(0), _CH)], cv0, semc0)

    def pair_body(i, carry):
        process(2 * i, 0)
        process(2 * i + 1, 1)
        return carry

    lax.fori_loop(0, _CPT // 2, pair_body, 0)

    dstv, srcv, av, bv, sa, sb = sets[0]
    pltpu.make_async_copy(a_hbm.at[dstv], av, sa).wait()
    pltpu.make_async_copy(b_hbm.at[srcv], bv, sb).wait()
    pltpu.make_async_copy(c_hbm.at[pl.ds(0, _CH)], cv0, semc0).wait()

    plsc.subcore_barrier()
    pltpu.sync_copy(s_sh.at[my_rows], out_s_hbm.at[cid, my_rows])

    @pl.when(sid == 0)
    def _():
        pltpu.sync_copy(cnt_sh, out_c_hbm.at[cid, 0])


def _build(interpret=False):
    node_pre = pl.pallas_call(
        _node_pre_body,
        grid=(_N // _NBLK,),
        in_specs=[
            pl.BlockSpec((_NBLK, _D), lambda i: (i, 0)),
            pl.BlockSpec((_D, 3 * _H), lambda i: (0, 0)),
        ],
        out_specs=pl.BlockSpec((_NBLK, 3 * _H), lambda i: (i, 0)),
        out_shape=jax.ShapeDtypeStruct((_N, 3 * _H), jnp.float32),
        interpret=interpret,
    )

    edge_pre = pl.pallas_call(
        _edge_pre_body,
        grid=(_EP // _EBLK,),
        in_specs=[
            pl.BlockSpec((_EBLK, _DE), lambda i: (i, 0)),
            pl.BlockSpec((_DE, _H), lambda i: (0, 0)),
            pl.BlockSpec((1, _H), lambda i: (0, 0)),
        ],
        out_specs=pl.BlockSpec((_EBLK, _H), lambda i: (i, 0)),
        out_shape=jax.ShapeDtypeStruct((_EP, _H), jnp.float32),
        interpret=interpret,
    )

    mesh = plsc.VectorSubcoreMesh(core_axis_name="c", subcore_axis_name="s")
    sc_edge = pl.kernel(
        _sc_edge_body,
        out_type=(
            jax.ShapeDtypeStruct((_NC, _NP, _D), jnp.float32),
            jax.ShapeDtypeStruct((_NC, 1, _NPC), jnp.float32),
        ),
        mesh=mesh,
        scratch_types=[
            pltpu.VMEM((_CH,), jnp.int32),
            pltpu.VMEM((_CH,), jnp.int32),
            pltpu.VMEM((_CH,), jnp.int32),
            pltpu.VMEM((_CH,), jnp.int32),
            pltpu.VMEM((_CH, _D), jnp.float32),
            pltpu.VMEM((_CH, _D), jnp.float32),
            pltpu.VMEM((_CH, _D), jnp.float32),
            pltpu.VMEM((_CH, _D), jnp.float32),
            pltpu.VMEM((_CH, _D), jnp.float32),
            pltpu.VMEM((_CH,), jnp.float32),
            pltpu.VMEM_SHARED((_NP, _D), jnp.float32),
            pltpu.VMEM_SHARED((_NPC,), jnp.float32),
            pltpu.SemaphoreType.DMA,
            pltpu.SemaphoreType.DMA,
            pltpu.SemaphoreType.DMA,
            pltpu.SemaphoreType.DMA,
            pltpu.SemaphoreType.DMA,
        ],
        interpret=interpret,
    )

    post = pl.pallas_call(
        _post_body,
        grid=(_N // _NBLK,),
        in_specs=[
            pl.BlockSpec((_NC, _NBLK, _D), lambda i: (0, i, 0)),
            pl.BlockSpec((_NBLK, _NC), lambda i: (i, 0)),
            pl.BlockSpec((_NBLK, _D), lambda i: (i, 0)),
            pl.BlockSpec((_D, _D), lambda i: (0, 0)),
            pl.BlockSpec((1, _D), lambda i: (0, 0)),
            pl.BlockSpec((1, _D), lambda i: (0, 0)),
            pl.BlockSpec((1, _D), lambda i: (0, 0)),
        ],
        out_specs=pl.BlockSpec((_NBLK, _D), lambda i: (i, 0)),
        out_shape=jax.ShapeDtypeStruct((_N, _D), jnp.float32),
        interpret=interpret,
    )

    @jax.jit
    def run(x, edge_index, edge_attr, W1, b1, W2, b2, Wskip, gamma, beta):
        dst = edge_index[0]
        src = edge_index[1]

        wn = jnp.concatenate([W1[:_D], W1[_D:2 * _D], Wskip], axis=1)
        nodes = node_pre(x, wn)
        a_tab = nodes[:, :_H]
        b_tab = nodes[:, _H:2 * _H]
        skip = nodes[:, 2 * _H:]
        pad_n = ((0, _NP - _N), (0, 0))
        a_tab = jnp.pad(a_tab, pad_n)
        b_tab = jnp.pad(b_tab, pad_n)

        dst_p = jnp.concatenate(
            [dst, jnp.full((_EP - _E,), _N, jnp.int32)])
        src_p = jnp.concatenate(
            [src, jnp.zeros((_EP - _E,), jnp.int32)])
        ea_p = jnp.pad(edge_attr, ((0, _EP - _E), (0, 0)))
        c_tab = edge_pre(ea_p, W1[2 * _D:], b1.reshape(1, _H))

        z2 = jnp.zeros((_RPS, _D), jnp.float32)
        z1 = jnp.zeros((_CPS,), jnp.float32)
        s_out, cnt_out = sc_edge(dst_p, src_p, c_tab, a_tab, b_tab, z2, z1)

        cnt_t = jnp.swapaxes(cnt_out.reshape(_NC, _NPC), 0, 1)
        out = post(s_out, cnt_t, skip, W2, b2.reshape(1, _D),
                   gamma.reshape(1, _D), beta.reshape(1, _D))
        return out

    return run


_impl = _build()


def kernel(x, edge_index, edge_attr, W1, b1, W2, b2, Wskip, gamma, beta):
    return _impl(x, edge_index, edge_attr, W1, b1, W2, b2, Wskip, gamma,
                 beta)
